# baseline (speedup 1.0000x reference)
import jax
import jax.numpy as jnp
from jax import lax
from jax.experimental import pallas as pl
from jax.experimental.pallas import tpu as pltpu
from jax.experimental.pallas import tpu_sc as plsc

NUM_CLASSES = 100000
EMBED_DIM = 64
BATCH = 16384

_INFO = plsc.get_sparse_core_info()
_NC = _INFO.num_cores
_NS = _INFO.num_subcores
_NW = _NC * _NS
_DIMS_PER_W = EMBED_DIM // _NW
_Q = BATCH // 4
_V = NUM_CLASSES + 1
_UNROLL = 16


def _emb_body(idx_hbm, tab_hbm, out_hbm, idx_v, row_v, out_a, out_b,
              sem_r, sem_i, sem_oa, sem_ob):
    wid = lax.axis_index("s") * _NC + lax.axis_index("c")
    out_bufs = (out_a, out_b)
    out_sems = (sem_oa, sem_ob)

    def gather_quarter(q, dst_out):
        @plsc.parallel_loop(0, _Q, step=16 * _UNROLL)
        def gbody(i):
            for u in range(_UNROLL):
                s = pl.ds(q * _Q + i + u * 16, 16)
                dst_out[pl.ds(i + u * 16, 16)] = (
                    plsc.load_gather(row_v, [idx_v[s]]))

    idx_cp = pltpu.async_copy(idx_hbm, idx_v, sem_i)
    out_waits = [None, None]
    idx_waited = False
    for d in range(_DIMS_PER_W):
        j = wid * _DIMS_PER_W + d
        row_cp = pltpu.async_copy(tab_hbm.at[j], row_v, sem_r)
        row_cp.wait()
        if not idx_waited:
            idx_cp.wait()
            idx_waited = True
        for q in range(4):
            b = q % 2
            if out_waits[b] is not None:
                out_waits[b].wait()
            gather_quarter(q, out_bufs[b])
            out_waits[b] = pltpu.async_copy(
                out_bufs[b], out_hbm.at[j, pl.ds(q * _Q, _Q)], out_sems[b])
    out_waits[0].wait()
    out_waits[1].wait()


@jax.jit
def _emb(class_labels, table_t):
    mesh = plsc.VectorSubcoreMesh(core_axis_name="c", subcore_axis_name="s")
    return pl.kernel(
        _emb_body,
        mesh=mesh,
        out_type=jax.ShapeDtypeStruct((EMBED_DIM, BATCH), jnp.float32),
        scratch_types=[
            pltpu.VMEM((BATCH,), jnp.int32),
            pltpu.VMEM((_V,), jnp.float32),
            pltpu.VMEM((_Q,), jnp.float32),
            pltpu.VMEM((_Q,), jnp.float32),
            pltpu.SemaphoreType.DMA,
            pltpu.SemaphoreType.DMA,
            pltpu.SemaphoreType.DMA,
            pltpu.SemaphoreType.DMA,
        ],
        compiler_params=pltpu.CompilerParams(
            use_tc_tiling_on_sc=True, needs_layout_passes=False),
    )(class_labels, table_t)


def kernel(class_labels, table):
    out_t = _emb(class_labels.astype(jnp.int32), table.T)
    return out_t.T


# speedup vs baseline: 1.1519x; 1.1519x over previous
import jax
import jax.numpy as jnp
from jax import lax
from jax.experimental import pallas as pl
from jax.experimental.pallas import tpu as pltpu
from jax.experimental.pallas import tpu_sc as plsc

NUM_CLASSES = 100000
EMBED_DIM = 64
BATCH = 16384

_INFO = plsc.get_sparse_core_info()
_NC = _INFO.num_cores
_NS = _INFO.num_subcores
_NW = _NC * _NS
_DIMS_PER_W = EMBED_DIM // _NW
_Q = BATCH // 4
_V = NUM_CLASSES + 1
_UNROLL = 4


def _emb_body(idx_hbm, tab_hbm, out_hbm, idx_v, row_v, out_a, out_b,
              sem_r, sem_i, sem_oa, sem_ob):
    wid = lax.axis_index("s") * _NC + lax.axis_index("c")
    out_bufs = (out_a, out_b)
    out_sems = (sem_oa, sem_ob)

    def gather_quarter(q, dst_out):
        @plsc.parallel_loop(0, _Q, step=16 * _UNROLL)
        def gbody(i):
            for u in range(_UNROLL):
                s = pl.ds(q * _Q + i + u * 16, 16)
                dst_out[pl.ds(i + u * 16, 16)] = (
                    plsc.load_gather(row_v, [idx_v[s]]))

    idx_cp = pltpu.async_copy(idx_hbm, idx_v, sem_i)
    out_waits = [None, None]
    idx_waited = False
    for d in range(_DIMS_PER_W):
        j = wid * _DIMS_PER_W + d
        row_cp = pltpu.async_copy(tab_hbm.at[j], row_v, sem_r)
        row_cp.wait()
        if not idx_waited:
            idx_cp.wait()
            idx_waited = True
        for q in range(4):
            b = q % 2
            if out_waits[b] is not None:
                out_waits[b].wait()
            gather_quarter(q, out_bufs[b])
            out_waits[b] = pltpu.async_copy(
                out_bufs[b], out_hbm.at[j, pl.ds(q * _Q, _Q)], out_sems[b])
    out_waits[0].wait()
    out_waits[1].wait()


@jax.jit
def _emb(class_labels, table_t):
    mesh = plsc.VectorSubcoreMesh(core_axis_name="c", subcore_axis_name="s")
    return pl.kernel(
        _emb_body,
        mesh=mesh,
        out_type=jax.ShapeDtypeStruct((EMBED_DIM, BATCH), jnp.float32),
        scratch_types=[
            pltpu.VMEM((BATCH,), jnp.int32),
            pltpu.VMEM((_V,), jnp.float32),
            pltpu.VMEM((_Q,), jnp.float32),
            pltpu.VMEM((_Q,), jnp.float32),
            pltpu.SemaphoreType.DMA,
            pltpu.SemaphoreType.DMA,
            pltpu.SemaphoreType.DMA,
            pltpu.SemaphoreType.DMA,
        ],
        compiler_params=pltpu.CompilerParams(
            use_tc_tiling_on_sc=True, needs_layout_passes=False),
    )(class_labels, table_t)


def kernel(class_labels, table):
    out_t = _emb(class_labels.astype(jnp.int32), table.T)
    return out_t.T
